# baseline (speedup 1.0000x reference)
import functools

import jax
import jax.numpy as jnp
from jax import lax
from jax.experimental import pallas as pl
from jax.experimental.pallas import tpu as pltpu
from jax.experimental.pallas import tpu_sc as plsc

_F32 = jnp.float32
_I32 = jnp.int32


_MESH = plsc.VectorSubcoreMesh(core_axis_name="c", subcore_axis_name="s")


@functools.partial(
    pl.kernel,
    mesh=_MESH,
    out_type=jax.ShapeDtypeStruct((128, 16), _F32),
    scratch_types=[
        pltpu.VMEM((4, 128), _F32),
        pltpu.VMEM((16,), _I32),
        pltpu.VMEM((16,), _F32),
        pltpu.VMEM((4, 16), _F32),
        pltpu.SemaphoreType.DMA,
    ],
)
def _sc_picks(lsc_hbm, ls1d_hbm, picks_hbm, rows_v, idx_v, vals_v, out_v, sem):
    wid = lax.axis_index("s") * 2 + lax.axis_index("c")
    i0 = wid * 4
    pltpu.sync_copy(lsc_hbm.at[pl.ds(i0, 4)], rows_v)

    lanes = lax.iota(_I32, 16)

    def _bcast_max_f32(x):
        for s in (1, 2, 4, 8):
            x = jnp.maximum(x, x.at[lanes ^ s].get(mode="promise_in_bounds"))
        return x

    def _bcast_min_i32(x):
        for s in (1, 2, 4, 8):
            x = jnp.minimum(x, x.at[lanes ^ s].get(mode="promise_in_bounds"))
        return x

    cvecs = []
    mvecs = []
    for t in range(4):
        bv = rows_v[t, pl.ds(0, 16)]
        bk = jnp.zeros((16,), _I32)
        for k in range(1, 8):
            v = rows_v[t, pl.ds(k * 16, 16)]
            upd = v > bv
            bv = jnp.where(upd, v, bv)
            bk = jnp.where(upd, k, bk)
        m_vec = _bcast_max_f32(bv)
        full_idx = bk * 16 + lanes
        c_vec = _bcast_min_i32(jnp.where(bv == m_vec, full_idx, 999))
        cvecs.append(c_vec)
        mvecs.append(m_vec)

    idxv = jnp.zeros((16,), _I32)
    for t in range(4):
        f_vec = cvecs[t] * 51 + ((i0 + t) * 6528 + 50)
        idxv = jnp.where(lanes == t, f_vec, idxv)
    idx_v[...] = idxv
    pltpu.async_copy(ls1d_hbm.at[idx_v], vals_v, sem).wait()

    vals = vals_v[...]
    neg_big = jnp.float32(-3.0e38)
    for t in range(4):
        sv_vec = _bcast_max_f32(jnp.where(lanes == t, vals, neg_big))
        row = jnp.where(lanes == 0, sv_vec, 0.0)
        row = jnp.where(lanes == 1, mvecs[t], row)
        out_v[t, :] = row
    pltpu.sync_copy(out_v, picks_hbm.at[pl.ds(i0, 4)])


def _tc_body(ls01_ref, lsc01_ref, fws_ref, fwsc_ref, picks_ref, gs_out, gsc_out):
    lsc01 = lsc01_ref[...]
    fws = fws_ref[...]
    fwsc = fwsc_ref[...]
    ls01_last = ls01_ref[:, :, 50]
    picks = picks_ref[...]

    iota_r = jax.lax.broadcasted_iota(_I32, (128, 128), 0)
    iota_c = jax.lax.broadcasted_iota(_I32, (128, 128), 1)

    row_sums = jnp.sum(fwsc, axis=1, keepdims=True)
    lsc0_col = jnp.transpose(lsc01[0:1, :])
    gsum0_col = jnp.concatenate([row_sums, row_sums], axis=0) + lsc0_col
    gsum0_row = jnp.transpose(gsum0_col)

    greater = (gsum0_row > gsum0_col) | ((gsum0_row == gsum0_col) & (iota_c < iota_r))
    rank128 = jnp.sum(greater.astype(_I32), axis=1, keepdims=True)
    selected_col = rank128 < 64
    selected_row = jnp.transpose(selected_col)

    fws2 = jnp.concatenate([fws, fws], axis=0)
    fwsc2 = jnp.concatenate([fwsc, fwsc], axis=0)
    lsl0_col = jnp.transpose(ls01_last[0:1, :])
    data = jnp.concatenate(
        [fws2, lsl0_col, fwsc2, lsc0_col, gsum0_col], axis=1)

    fws_t = jnp.transpose(fws)
    r64 = jnp.zeros((64, 64), _F32)
    nd64 = jnp.ones((64, 64), _F32)
    for c in range(51):
        a_c = fws[:, c:c + 1]
        b_c = fws_t[c:c + 1, :]
        r64 = r64 + nd64 * jnp.sign(a_c - b_c)
        nd64 = nd64 * (a_c == b_c).astype(_F32)
    less64 = (r64 < 0).astype(_F32)
    less64_2 = jnp.concatenate([less64, less64], axis=0)
    less64_4 = jnp.concatenate([less64_2, less64_2], axis=1)
    lessd = (lsl0_col < jnp.transpose(lsl0_col)).astype(_F32)
    same_m = ((iota_r % 64) == (iota_c % 64)).astype(_F32)
    less = same_m * lessd + (1.0 - same_m) * less64_4

    rank_sel = jnp.sum(less * selected_col.astype(_F32), axis=0,
                       keepdims=True).astype(_I32)
    iota64_r = jax.lax.broadcasted_iota(_I32, (64, 128), 0)
    p_mat = ((rank_sel == iota64_r) & selected_row).astype(_F32)
    gs_sorted = jax.lax.dot(p_mat, data[:, 0:52],
                            precision=jax.lax.Precision.HIGHEST)
    gsc_sorted = jax.lax.dot(p_mat, data[:, 52:104],
                             precision=jax.lax.Precision.HIGHEST)
    gsum_sorted = jax.lax.dot(p_mat, data[:, 104:105],
                              precision=jax.lax.Precision.HIGHEST)

    gsum_sorted_row = jnp.transpose(gsum_sorted)
    lsc1_2x64 = jnp.concatenate([lsc01[1:2, 0:64], lsc01[1:2, 64:128]], axis=0)
    v1 = gsum_sorted_row + lsc1_2x64
    idxj = (jax.lax.broadcasted_iota(_I32, (2, 64), 0) * 64
            + jax.lax.broadcasted_iota(_I32, (2, 64), 1))
    v1_max = jnp.max(v1)
    c1 = jnp.min(jnp.where(v1 == v1_max, idxj, 999))
    m1 = c1 % 64

    iota64_c1 = jax.lax.broadcasted_iota(_I32, (64, 1), 0)
    e_col = (iota64_c1 == m1).astype(_F32)
    prefix_gs = jnp.sum(e_col * gs_sorted, axis=0, keepdims=True)
    prefix_gsc = jnp.sum(e_col * gsc_sorted, axis=0, keepdims=True)

    iota128_row = jax.lax.broadcasted_iota(_I32, (1, 128), 1)
    sel_c1 = (iota128_row == c1).astype(_F32)
    lsl1_c1 = jnp.sum(sel_c1 * ls01_last[1:2, :], axis=1, keepdims=True)
    lsc1_c1 = jnp.sum(sel_c1 * lsc01[1:2, :], axis=1, keepdims=True)

    picks_struct_row = jnp.transpose(picks[:, 0:1])
    picks_score_row = jnp.transpose(picks[:, 1:2])

    row_gs = jnp.concatenate(
        [prefix_gs, lsl1_c1, picks_struct_row[:, 2:128]], axis=1)
    row_gsc = jnp.concatenate(
        [prefix_gsc, lsc1_c1, picks_score_row[:, 2:128]], axis=1)
    gs_out[...] = jnp.broadcast_to(row_gs, (64, 179))
    gsc_out[...] = jnp.broadcast_to(row_gsc, (64, 179))


def kernel(local_structs, local_scores, first_window_struct, first_window_scores):
    picks = _sc_picks(local_scores, local_structs.reshape(-1))
    out_shape = (
        jax.ShapeDtypeStruct((64, 179), _F32),
        jax.ShapeDtypeStruct((64, 179), _F32),
    )
    return pl.pallas_call(_tc_body, out_shape=out_shape)(
        local_structs[0:2], local_scores[0:2],
        first_window_struct, first_window_scores, picks)
